# baseline (speedup 1.0000x reference)
import functools

import jax
import jax.numpy as jnp
from jax import lax
from jax.experimental import pallas as pl
from jax.experimental.pallas import tpu as pltpu
from jax.experimental.pallas import tpu_sc as plsc

_N = 10000
_E = 320000
_EPS = 1e-5
_NW = 32
_EPW = _E // _NW
_CH = 80
_BM = 1000


@functools.lru_cache(maxsize=None)
def _make_cmm(n_add, fin, fout, emit_x, emit_ycat=False):
    def body(*refs):
        a_refs = refs[:n_add]
        w_ref, b_ref = refs[n_add], refs[n_add + 1]
        out_refs = refs[n_add + 2:]
        x = a_refs[0][...]
        for r in a_refs[1:]:
            x = x + r[...]
        y = jnp.dot(x, w_ref[...], preferred_element_type=jnp.float32)
        if emit_x:
            out_refs[0][...] = x
        out_refs[-2][...] = y if emit_ycat else y[:, :fout]
        out_refs[-1][...] = y[:, fout:] + b_ref[...]

    sup_w = 2 * fout if emit_ycat else fout
    in_specs = [pl.BlockSpec((_BM, fin), lambda i: (i, 0)) for _ in range(n_add)]
    in_specs.append(pl.BlockSpec((fin, 2 * fout), lambda i: (0, 0)))
    in_specs.append(pl.BlockSpec((1, fout), lambda i: (0, 0)))
    out_specs = []
    out_shape = []
    if emit_x:
        out_specs.append(pl.BlockSpec((_BM, fin), lambda i: (i, 0)))
        out_shape.append(jax.ShapeDtypeStruct((_N, fin), jnp.float32))
    out_specs += [pl.BlockSpec((_BM, sup_w), lambda i: (i, 0)),
                  pl.BlockSpec((_BM, fout), lambda i: (i, 0))]
    out_shape += [jax.ShapeDtypeStruct((_N, sup_w), jnp.float32),
                  jax.ShapeDtypeStruct((_N, fout), jnp.float32)]

    return pl.pallas_call(
        body,
        grid=(_N // _BM,),
        in_specs=in_specs,
        out_specs=out_specs,
        out_shape=out_shape,
    )


def _out_body(p0_ref, p1_ref, base_ref, o_ref):
    fout = base_ref.shape[1]
    z = p0_ref[:, :fout] + p1_ref[:, :fout] + base_ref[...]
    m = jnp.max(z, axis=1, keepdims=True)
    e = jnp.exp(z - m)
    lse = jnp.log(jnp.sum(e, axis=1, keepdims=True)) + m
    o_ref[...] = z - lse


@functools.lru_cache(maxsize=None)
def _make_out(fout):
    return pl.pallas_call(
        _out_body,
        grid=(_N // _BM,),
        in_specs=[pl.BlockSpec((_BM, 2 * fout), lambda i: (i, 0))] * 2
        + [pl.BlockSpec((_BM, fout), lambda i: (i, 0))],
        out_specs=pl.BlockSpec((_BM, fout), lambda i: (i, 0)),
        out_shape=jax.ShapeDtypeStruct((_N, fout), jnp.float32),
    )


_GDN = lax.GatherDimensionNumbers(
    offset_dims=(), collapsed_slice_dims=(0,), start_index_map=(0,))


@functools.lru_cache(maxsize=None)
def _make_spmm(d):
    nchunks = _EPW // _CH
    rps, rps_last = 632, _N - 15 * 632
    mesh = plsc.VectorSubcoreMesh(core_axis_name="c", subcore_axis_name="s")

    @functools.partial(
        pl.kernel,
        out_type=jax.ShapeDtypeStruct((2, _N, d), jnp.float32),
        mesh=mesh,
        scratch_types=[
            [pltpu.VMEM((_CH, d), jnp.float32)] * 4,
            [pltpu.VMEM((_CH,), jnp.int32)] * 4,
            [pltpu.VMEM((_CH,), jnp.int32)] * 4,
            [pltpu.VMEM((_CH,), jnp.float32)] * 4,
            pltpu.VMEM_SHARED((_N, d), jnp.float32),
            [pltpu.SemaphoreType.DMA] * 4,
            [pltpu.SemaphoreType.DMA] * 4,
            [pltpu.SemaphoreType.DMA] * 4,
            [pltpu.SemaphoreType.DMA] * 4,
            [pltpu.SemaphoreType.DMA] * 4,
        ],
    )
    def spmm(sup_hbm, src_hbm, dst_hbm, w_hbm, zero_hbm, out_hbm,
             rows, srcv, dstv, wv, acc_sh, gsem, ssem, srcsem, dsem, wsem):
        c = lax.axis_index("c")
        s = lax.axis_index("s")
        wid = s * 2 + c
        ebase = wid * _EPW

        def idx_start(i, b):
            eb = ebase + i * _CH
            pltpu.async_copy(src_hbm.at[pl.ds(eb, _CH)], srcv[b], srcsem[b])

        def idx_wait(b):
            pltpu.make_async_copy(src_hbm.at[pl.ds(0, _CH)], srcv[b],
                                  srcsem[b]).wait()

        def dstw_start(i, b):
            eb = ebase + i * _CH
            pltpu.async_copy(dst_hbm.at[pl.ds(eb, _CH)], dstv[b], dsem[b])
            pltpu.async_copy(w_hbm.at[pl.ds(eb, _CH)], wv[b], wsem[b])

        def dstw_wait(b):
            pltpu.make_async_copy(dst_hbm.at[pl.ds(0, _CH)], dstv[b],
                                  dsem[b]).wait()
            pltpu.make_async_copy(w_hbm.at[pl.ds(0, _CH)], wv[b],
                                  wsem[b]).wait()

        def gather_start(b):
            pltpu.async_copy(sup_hbm.at[srcv[b]], rows[b], gsem[b])

        def gather_wait(b):
            pltpu.make_async_copy(sup_hbm.at[srcv[b]], rows[b], gsem[b]).wait()

        def scatter_start(b):
            pltpu.async_copy(rows[b], acc_sh.at[dstv[b]], ssem[b], add=True)

        def scatter_drain(b):
            pltpu.make_async_copy(rows[b], acc_sh.at[dstv[b]], ssem[b]).wait()

        def stage_scale(b):
            rows_v = rows[b]
            for g in range(_CH // 16):
                w16 = wv[b][pl.ds(g * 16, 16)]
                for j in range(16):
                    e = g * 16 + j
                    wb = lax.gather(
                        w16, jnp.full((16, 1), j, jnp.int32), _GDN,
                        slice_sizes=(1,),
                        mode=lax.GatherScatterMode.PROMISE_IN_BOUNDS)
                    for cb in range(d // 16):
                        sl = pl.ds(cb * 16, 16)
                        rows_v[e, sl] = rows_v[e, sl] * wb

        @pl.when(s < 15)
        def _():
            pltpu.sync_copy(zero_hbm.at[pl.ds(s * rps, rps)],
                            acc_sh.at[pl.ds(s * rps, rps)])

        @pl.when(s == 15)
        def _():
            pltpu.sync_copy(zero_hbm.at[pl.ds(15 * rps, rps_last)],
                            acc_sh.at[pl.ds(15 * rps, rps_last)])

        def slot(i, b):
            b3 = (b + 3) % 4
            gather_wait(b)

            @pl.when(i >= 1)
            def _():
                scatter_drain(b3)

            @pl.when(i + 4 < nchunks)
            def _():
                idx_start(i + 4, b)

            @pl.when(i + 3 < nchunks)
            def _():
                dstw_start(i + 3, b3)
                idx_wait(b3)
                gather_start(b3)

            dstw_wait(b)
            stage_scale(b)
            scatter_start(b)

        for k in range(4):
            idx_start(k, k)
        for k in range(3):
            dstw_start(k, k)
        for k in range(3):
            idx_wait(k)
            gather_start(k)
        plsc.subcore_barrier()

        @pl.loop(0, nchunks - 1, step=4)
        def _(ci):
            for k in range(4):
                slot(ci + k, k)

        slot(nchunks - 1, (nchunks - 1) % 4)
        scatter_drain((nchunks - 1) % 4)
        plsc.subcore_barrier()

        @pl.when(s < 15)
        def _():
            pltpu.sync_copy(acc_sh.at[pl.ds(s * rps, rps)],
                            out_hbm.at[c, pl.ds(s * rps, rps)])

        @pl.when(s == 15)
        def _():
            pltpu.sync_copy(acc_sh.at[pl.ds(15 * rps, rps_last)],
                            out_hbm.at[c, pl.ds(15 * rps, rps_last)])

    return spmm


def _fold(W, S, b, g, bt):
    gg = g * (1.0 / jnp.sqrt(1.0 + _EPS))
    return jnp.concatenate([W * gg[None, :], S * gg[None, :]], axis=1), \
        (b * gg + bt).reshape(1, -1)


def kernel(fea, edge_index, edge_weight,
           W0, S0, b0, g0, bt0,
           W1, S1, b1, g1, bt1,
           W2, S2, b2, g2, bt2,
           W3, S3, b3, g3, bt3):
    src = edge_index[0]
    dst = edge_index[1]
    wc0, bf0 = _fold(W0, S0, b0, g0, bt0)
    wc1, bf1 = _fold(W1, S1, b1, g1, bt1)
    wc2, bf2 = _fold(W2, S2, b2, g2, bt2)
    wc3, bf3 = _fold(W3, S3, b3, g3, bt3)
    z128 = jnp.zeros((_N, 128), jnp.float32)

    spmm128 = _make_spmm(128)

    sup0, base0 = _make_cmm(1, 128, 128, False)(fea, wc0, bf0)
    p0 = spmm128(sup0, src, dst, edge_weight, z128)
    x1, sup1, base1 = _make_cmm(3, 128, 128, True)(p0[0], p0[1], base0, wc1, bf1)
    p1 = spmm128(sup1, src, dst, edge_weight, z128)
    sup2, base2 = _make_cmm(3, 128, 128, False)(p1[0], p1[1], base1, wc2, bf2)
    p2 = spmm128(sup2, src, dst, edge_weight, z128)
    ycat3, base3 = _make_cmm(4, 128, 64, False, True)(p2[0], p2[1], base2, x1,
                                                      wc3, bf3)
    p3 = spmm128(ycat3, src, dst, edge_weight, z128)
    return _make_out(64)(p3[0], p3[1], base3)


# speedup vs baseline: 1.2130x; 1.2130x over previous
import functools

import jax
import jax.numpy as jnp
from jax import lax
from jax.experimental import pallas as pl
from jax.experimental.pallas import tpu as pltpu
from jax.experimental.pallas import tpu_sc as plsc

_N = 10000
_E = 320000
_EPS = 1e-5
_NW = 32
_EPW = _E // _NW
_CH = 64
_TAIL = _EPW - (_EPW // _CH) * _CH
_BM = 1000


@functools.lru_cache(maxsize=None)
def _make_cmm(n_add, fin, fout, emit_x, emit_ycat=False):
    def body(*refs):
        a_refs = refs[:n_add]
        w_ref, b_ref = refs[n_add], refs[n_add + 1]
        out_refs = refs[n_add + 2:]
        x = a_refs[0][...]
        for r in a_refs[1:]:
            x = x + r[...]
        y = jnp.dot(x, w_ref[...], preferred_element_type=jnp.float32)
        if emit_x:
            out_refs[0][...] = x
        out_refs[-2][...] = y if emit_ycat else y[:, :fout]
        out_refs[-1][...] = y[:, fout:] + b_ref[...]

    sup_w = 2 * fout if emit_ycat else fout
    in_specs = [pl.BlockSpec((_BM, fin), lambda i: (i, 0)) for _ in range(n_add)]
    in_specs.append(pl.BlockSpec((fin, 2 * fout), lambda i: (0, 0)))
    in_specs.append(pl.BlockSpec((1, fout), lambda i: (0, 0)))
    out_specs = []
    out_shape = []
    if emit_x:
        out_specs.append(pl.BlockSpec((_BM, fin), lambda i: (i, 0)))
        out_shape.append(jax.ShapeDtypeStruct((_N, fin), jnp.float32))
    out_specs += [pl.BlockSpec((_BM, sup_w), lambda i: (i, 0)),
                  pl.BlockSpec((_BM, fout), lambda i: (i, 0))]
    out_shape += [jax.ShapeDtypeStruct((_N, sup_w), jnp.float32),
                  jax.ShapeDtypeStruct((_N, fout), jnp.float32)]

    return pl.pallas_call(
        body,
        grid=(_N // _BM,),
        in_specs=in_specs,
        out_specs=out_specs,
        out_shape=out_shape,
    )


def _out_body(p0_ref, p1_ref, base_ref, o_ref):
    fout = base_ref.shape[1]
    z = p0_ref[:, :fout] + p1_ref[:, :fout] + base_ref[...]
    m = jnp.max(z, axis=1, keepdims=True)
    e = jnp.exp(z - m)
    lse = jnp.log(jnp.sum(e, axis=1, keepdims=True)) + m
    o_ref[...] = z - lse


@functools.lru_cache(maxsize=None)
def _make_out(fout):
    return pl.pallas_call(
        _out_body,
        grid=(_N // _BM,),
        in_specs=[pl.BlockSpec((_BM, 2 * fout), lambda i: (i, 0))] * 2
        + [pl.BlockSpec((_BM, fout), lambda i: (i, 0))],
        out_specs=pl.BlockSpec((_BM, fout), lambda i: (i, 0)),
        out_shape=jax.ShapeDtypeStruct((_N, fout), jnp.float32),
    )


_GDN = lax.GatherDimensionNumbers(
    offset_dims=(), collapsed_slice_dims=(0,), start_index_map=(0,))


@functools.lru_cache(maxsize=None)
def _make_spmm(d):
    nchunks = _EPW // _CH
    rps, rps_last = 632, _N - 15 * 632
    mesh = plsc.VectorSubcoreMesh(core_axis_name="c", subcore_axis_name="s")

    @functools.partial(
        pl.kernel,
        out_type=jax.ShapeDtypeStruct((2, _N, d), jnp.float32),
        mesh=mesh,
        scratch_types=[
            [pltpu.VMEM((_CH, d), jnp.float32)] * 5,
            [pltpu.VMEM((_CH,), jnp.int32)] * 5,
            [pltpu.VMEM((_CH,), jnp.int32)] * 5,
            [pltpu.VMEM((_CH,), jnp.float32)] * 5,
            pltpu.VMEM((_TAIL,), jnp.int32),
            pltpu.VMEM((_TAIL,), jnp.int32),
            pltpu.VMEM((_TAIL,), jnp.float32),
            pltpu.VMEM_SHARED((_N, d), jnp.float32),
            [pltpu.SemaphoreType.DMA] * 5,
            [pltpu.SemaphoreType.DMA] * 5,
            [pltpu.SemaphoreType.DMA] * 5,
            [pltpu.SemaphoreType.DMA] * 5,
            [pltpu.SemaphoreType.DMA] * 5,
        ],
    )
    def spmm(sup_hbm, src_hbm, dst_hbm, w_hbm, zero_hbm, out_hbm,
             rows, srcv, dstv, wv, tsrc, tdst, tw, acc_sh,
             gsem, ssem, srcsem, dsem, wsem):
        c = lax.axis_index("c")
        s = lax.axis_index("s")
        wid = s * 2 + c
        ebase = wid * _EPW

        def idx_start(i, b):
            eb = ebase + i * _CH
            pltpu.async_copy(src_hbm.at[pl.ds(eb, _CH)], srcv[b], srcsem[b])

        def idx_wait(b):
            pltpu.make_async_copy(src_hbm.at[pl.ds(0, _CH)], srcv[b],
                                  srcsem[b]).wait()

        def dstw_start(i, b):
            eb = ebase + i * _CH
            pltpu.async_copy(dst_hbm.at[pl.ds(eb, _CH)], dstv[b], dsem[b])
            pltpu.async_copy(w_hbm.at[pl.ds(eb, _CH)], wv[b], wsem[b])

        def dstw_wait(b):
            pltpu.make_async_copy(dst_hbm.at[pl.ds(0, _CH)], dstv[b],
                                  dsem[b]).wait()
            pltpu.make_async_copy(w_hbm.at[pl.ds(0, _CH)], wv[b],
                                  wsem[b]).wait()

        def gather_start(b):
            pltpu.async_copy(sup_hbm.at[srcv[b]], rows[b], gsem[b])

        def gather_wait(b):
            pltpu.make_async_copy(sup_hbm.at[srcv[b]], rows[b], gsem[b]).wait()

        def scatter_start(b):
            pltpu.async_copy(rows[b], acc_sh.at[dstv[b]], ssem[b], add=True)

        def scatter_drain(b):
            pltpu.make_async_copy(rows[b], acc_sh.at[dstv[b]], ssem[b]).wait()

        def stage_scale(b):
            rows_v = rows[b]
            for g in range(_CH // 16):
                w16 = wv[b][pl.ds(g * 16, 16)]
                for j in range(16):
                    e = g * 16 + j
                    wb = lax.gather(
                        w16, jnp.full((16, 1), j, jnp.int32), _GDN,
                        slice_sizes=(1,),
                        mode=lax.GatherScatterMode.PROMISE_IN_BOUNDS)
                    for cb in range(d // 16):
                        sl = pl.ds(cb * 16, 16)
                        rows_v[e, sl] = rows_v[e, sl] * wb

        @pl.when(s < 15)
        def _():
            pltpu.sync_copy(zero_hbm.at[pl.ds(s * rps, rps)],
                            acc_sh.at[pl.ds(s * rps, rps)])

        @pl.when(s == 15)
        def _():
            pltpu.sync_copy(zero_hbm.at[pl.ds(15 * rps, rps_last)],
                            acc_sh.at[pl.ds(15 * rps, rps_last)])

        def slot(i, b):
            b3 = (b + 3) % 5
            gather_wait(b)

            @pl.when(i >= 2)
            def _():
                scatter_drain(b3)

            @pl.when(i + 5 < nchunks)
            def _():
                idx_start(i + 5, b)

            @pl.when(i + 3 < nchunks)
            def _():
                dstw_start(i + 3, b3)
                idx_wait(b3)
                gather_start(b3)

            dstw_wait(b)
            stage_scale(b)
            scatter_start(b)

        for k in range(5):
            idx_start(k, k)
        for k in range(3):
            dstw_start(k, k)
        for k in range(3):
            idx_wait(k)
            gather_start(k)
        plsc.subcore_barrier()

        @pl.loop(0, nchunks - 1, step=5)
        def _(ci):
            for k in range(5):
                slot(ci + k, k)

        slot(nchunks - 1, (nchunks - 1) % 5)
        scatter_drain((nchunks - 2) % 5)
        scatter_drain((nchunks - 1) % 5)

        teb = ebase + nchunks * _CH
        pltpu.sync_copy(src_hbm.at[pl.ds(teb, _TAIL)], tsrc)
        pltpu.sync_copy(dst_hbm.at[pl.ds(teb, _TAIL)], tdst)
        pltpu.sync_copy(w_hbm.at[pl.ds(teb, _TAIL)], tw)
        pltpu.async_copy(sup_hbm.at[tsrc], rows[0].at[pl.ds(0, _TAIL)],
                         gsem[0]).wait()
        tw16 = tw[...]
        for j in range(_TAIL):
            wb = lax.gather(
                tw16, jnp.full((16, 1), j, jnp.int32), _GDN,
                slice_sizes=(1,),
                mode=lax.GatherScatterMode.PROMISE_IN_BOUNDS)
            for cb in range(d // 16):
                sl = pl.ds(cb * 16, 16)
                rows[0][j, sl] = rows[0][j, sl] * wb
        pltpu.async_copy(rows[0].at[pl.ds(0, _TAIL)], acc_sh.at[tdst],
                         ssem[0], add=True).wait()
        plsc.subcore_barrier()

        @pl.when(s < 15)
        def _():
            pltpu.sync_copy(acc_sh.at[pl.ds(s * rps, rps)],
                            out_hbm.at[c, pl.ds(s * rps, rps)])

        @pl.when(s == 15)
        def _():
            pltpu.sync_copy(acc_sh.at[pl.ds(15 * rps, rps_last)],
                            out_hbm.at[c, pl.ds(15 * rps, rps_last)])

    return spmm


def _fold(W, S, b, g, bt):
    gg = g * (1.0 / jnp.sqrt(1.0 + _EPS))
    return jnp.concatenate([W * gg[None, :], S * gg[None, :]], axis=1), \
        (b * gg + bt).reshape(1, -1)


def kernel(fea, edge_index, edge_weight,
           W0, S0, b0, g0, bt0,
           W1, S1, b1, g1, bt1,
           W2, S2, b2, g2, bt2,
           W3, S3, b3, g3, bt3):
    src = edge_index[0]
    dst = edge_index[1]
    wc0, bf0 = _fold(W0, S0, b0, g0, bt0)
    wc1, bf1 = _fold(W1, S1, b1, g1, bt1)
    wc2, bf2 = _fold(W2, S2, b2, g2, bt2)
    wc3, bf3 = _fold(W3, S3, b3, g3, bt3)
    z128 = jnp.zeros((_N, 128), jnp.float32)

    spmm128 = _make_spmm(128)

    sup0, base0 = _make_cmm(1, 128, 128, False)(fea, wc0, bf0)
    p0 = spmm128(sup0, src, dst, edge_weight, z128)
    x1, sup1, base1 = _make_cmm(3, 128, 128, True)(p0[0], p0[1], base0, wc1, bf1)
    p1 = spmm128(sup1, src, dst, edge_weight, z128)
    sup2, base2 = _make_cmm(3, 128, 128, False)(p1[0], p1[1], base1, wc2, bf2)
    p2 = spmm128(sup2, src, dst, edge_weight, z128)
    ycat3, base3 = _make_cmm(4, 128, 64, False, True)(p2[0], p2[1], base2, x1,
                                                      wc3, bf3)
    p3 = spmm128(ycat3, src, dst, edge_weight, z128)
    return _make_out(64)(p3[0], p3[1], base3)


# speedup vs baseline: 1.3631x; 1.1238x over previous
import functools

import jax
import jax.numpy as jnp
from jax import lax
from jax.experimental import pallas as pl
from jax.experimental.pallas import tpu as pltpu
from jax.experimental.pallas import tpu_sc as plsc

_N = 10000
_E = 320000
_EPS = 1e-5
_NW = 32
_EPW = _E // _NW
_CH = 80
_BM = 1000


@functools.lru_cache(maxsize=None)
def _make_cmm(n_add, fin, fout, emit_x, emit_ycat=False):
    def body(*refs):
        a_refs = refs[:n_add]
        w_ref, b_ref = refs[n_add], refs[n_add + 1]
        out_refs = refs[n_add + 2:]
        x = a_refs[0][...]
        for r in a_refs[1:]:
            x = x + r[...]
        y = jnp.dot(x, w_ref[...], preferred_element_type=jnp.float32)
        if emit_x:
            out_refs[0][...] = x
        out_refs[-2][...] = y if emit_ycat else y[:, :fout]
        out_refs[-1][...] = y[:, fout:] + b_ref[...]

    sup_w = 2 * fout if emit_ycat else fout
    in_specs = [pl.BlockSpec((_BM, fin), lambda i: (i, 0)) for _ in range(n_add)]
    in_specs.append(pl.BlockSpec((fin, 2 * fout), lambda i: (0, 0)))
    in_specs.append(pl.BlockSpec((1, fout), lambda i: (0, 0)))
    out_specs = []
    out_shape = []
    if emit_x:
        out_specs.append(pl.BlockSpec((_BM, fin), lambda i: (i, 0)))
        out_shape.append(jax.ShapeDtypeStruct((_N, fin), jnp.float32))
    out_specs += [pl.BlockSpec((_BM, sup_w), lambda i: (i, 0)),
                  pl.BlockSpec((_BM, fout), lambda i: (i, 0))]
    out_shape += [jax.ShapeDtypeStruct((_N, sup_w), jnp.float32),
                  jax.ShapeDtypeStruct((_N, fout), jnp.float32)]

    return pl.pallas_call(
        body,
        grid=(_N // _BM,),
        in_specs=in_specs,
        out_specs=out_specs,
        out_shape=out_shape,
    )


def _out_body(p0_ref, p1_ref, base_ref, o_ref):
    fout = base_ref.shape[1]
    z = p0_ref[:, :fout] + p1_ref[:, :fout] + base_ref[...]
    m = jnp.max(z, axis=1, keepdims=True)
    e = jnp.exp(z - m)
    lse = jnp.log(jnp.sum(e, axis=1, keepdims=True)) + m
    o_ref[...] = z - lse


@functools.lru_cache(maxsize=None)
def _make_out(fout):
    return pl.pallas_call(
        _out_body,
        grid=(_N // _BM,),
        in_specs=[pl.BlockSpec((_BM, 2 * fout), lambda i: (i, 0))] * 2
        + [pl.BlockSpec((_BM, fout), lambda i: (i, 0))],
        out_specs=pl.BlockSpec((_BM, fout), lambda i: (i, 0)),
        out_shape=jax.ShapeDtypeStruct((_N, fout), jnp.float32),
    )


_GDN = lax.GatherDimensionNumbers(
    offset_dims=(), collapsed_slice_dims=(0,), start_index_map=(0,))


@functools.lru_cache(maxsize=None)
def _make_spmm(d, d_scale):
    nchunks = _EPW // _CH
    rps, rps_last = 632, _N - 15 * 632
    mesh = plsc.VectorSubcoreMesh(core_axis_name="c", subcore_axis_name="s")

    @functools.partial(
        pl.kernel,
        out_type=jax.ShapeDtypeStruct((2, _N, d), jnp.float32),
        mesh=mesh,
        scratch_types=[
            [pltpu.VMEM((_CH, d), jnp.float32)] * 4,
            [pltpu.VMEM((_CH,), jnp.int32)] * 4,
            [pltpu.VMEM((_CH,), jnp.int32)] * 4,
            [pltpu.VMEM((_CH,), jnp.float32)] * 4,
            pltpu.VMEM_SHARED((_N, d), jnp.float32),
            [pltpu.SemaphoreType.DMA] * 4,
            [pltpu.SemaphoreType.DMA] * 4,
            [pltpu.SemaphoreType.DMA] * 4,
            [pltpu.SemaphoreType.DMA] * 4,
            [pltpu.SemaphoreType.DMA] * 4,
        ],
    )
    def spmm(sup_hbm, src_hbm, dst_hbm, w_hbm, zero_hbm, out_hbm,
             rows, srcv, dstv, wv, acc_sh, gsem, ssem, srcsem, dsem, wsem):
        c = lax.axis_index("c")
        s = lax.axis_index("s")
        wid = s * 2 + c
        ebase = wid * _EPW

        def idx_start(i, b):
            eb = ebase + i * _CH
            pltpu.async_copy(src_hbm.at[pl.ds(eb, _CH)], srcv[b], srcsem[b])

        def idx_wait(b):
            pltpu.make_async_copy(src_hbm.at[pl.ds(0, _CH)], srcv[b],
                                  srcsem[b]).wait()

        def dstw_start(i, b):
            eb = ebase + i * _CH
            pltpu.async_copy(dst_hbm.at[pl.ds(eb, _CH)], dstv[b], dsem[b])
            pltpu.async_copy(w_hbm.at[pl.ds(eb, _CH)], wv[b], wsem[b])

        def dstw_wait(b):
            pltpu.make_async_copy(dst_hbm.at[pl.ds(0, _CH)], dstv[b],
                                  dsem[b]).wait()
            pltpu.make_async_copy(w_hbm.at[pl.ds(0, _CH)], wv[b],
                                  wsem[b]).wait()

        def gather_start(b):
            pltpu.async_copy(sup_hbm.at[srcv[b]], rows[b], gsem[b])

        def gather_wait(b):
            pltpu.make_async_copy(sup_hbm.at[srcv[b]], rows[b], gsem[b]).wait()

        def scatter_start(b):
            pltpu.async_copy(rows[b], acc_sh.at[dstv[b]], ssem[b], add=True)

        def scatter_drain(b):
            pltpu.make_async_copy(rows[b], acc_sh.at[dstv[b]], ssem[b]).wait()

        def stage_scale(b):
            rows_v = rows[b]
            for g in range(_CH // 16):
                w16 = wv[b][pl.ds(g * 16, 16)]
                for j in range(16):
                    e = g * 16 + j
                    wb = lax.gather(
                        w16, jnp.full((16, 1), j, jnp.int32), _GDN,
                        slice_sizes=(1,),
                        mode=lax.GatherScatterMode.PROMISE_IN_BOUNDS)
                    for cb in range(d_scale // 16):
                        sl = pl.ds(cb * 16, 16)
                        rows_v[e, sl] = rows_v[e, sl] * wb

        @pl.when(s < 15)
        def _():
            pltpu.sync_copy(zero_hbm.at[pl.ds(s * rps, rps)],
                            acc_sh.at[pl.ds(s * rps, rps)])

        @pl.when(s == 15)
        def _():
            pltpu.sync_copy(zero_hbm.at[pl.ds(15 * rps, rps_last)],
                            acc_sh.at[pl.ds(15 * rps, rps_last)])

        def slot(i, b):
            b2, b3 = (b + 2) % 4, (b + 3) % 4

            @pl.when(i >= 2)
            def _():
                scatter_drain(b2)

            @pl.when(i + 3 < nchunks)
            def _():
                idx_start(i + 3, b3)

            @pl.when(i + 2 < nchunks)
            def _():
                dstw_start(i + 2, b2)
                idx_wait(b2)
                gather_start(b2)

            gather_wait(b)
            dstw_wait(b)
            stage_scale(b)
            scatter_start(b)

        idx_start(0, 0)
        idx_start(1, 1)
        idx_start(2, 2)
        dstw_start(0, 0)
        dstw_start(1, 1)
        idx_wait(0)
        gather_start(0)
        idx_wait(1)
        gather_start(1)
        plsc.subcore_barrier()

        @pl.loop(0, nchunks - 1, step=4)
        def _(ci):
            for k in range(4):
                slot(ci + k, k)

        slot(nchunks - 1, (nchunks - 1) % 4)
        scatter_drain((nchunks - 2) % 4)
        scatter_drain((nchunks - 1) % 4)
        plsc.subcore_barrier()

        @pl.when(s < 15)
        def _():
            pltpu.sync_copy(acc_sh.at[pl.ds(s * rps, rps)],
                            out_hbm.at[c, pl.ds(s * rps, rps)])

        @pl.when(s == 15)
        def _():
            pltpu.sync_copy(acc_sh.at[pl.ds(15 * rps, rps_last)],
                            out_hbm.at[c, pl.ds(15 * rps, rps_last)])

    return spmm


def _fold(W, S, b, g, bt):
    gg = g * (1.0 / jnp.sqrt(1.0 + _EPS))
    return jnp.concatenate([W * gg[None, :], S * gg[None, :]], axis=1), \
        (b * gg + bt).reshape(1, -1)


def kernel(fea, edge_index, edge_weight,
           W0, S0, b0, g0, bt0,
           W1, S1, b1, g1, bt1,
           W2, S2, b2, g2, bt2,
           W3, S3, b3, g3, bt3):
    src = edge_index[0]
    dst = edge_index[1]
    wc0, bf0 = _fold(W0, S0, b0, g0, bt0)
    wc1, bf1 = _fold(W1, S1, b1, g1, bt1)
    wc2, bf2 = _fold(W2, S2, b2, g2, bt2)
    wc3, bf3 = _fold(W3, S3, b3, g3, bt3)
    z128 = jnp.zeros((_N, 128), jnp.float32)

    spmm128 = _make_spmm(128, 128)

    sup0, base0 = _make_cmm(1, 128, 128, False)(fea, wc0, bf0)
    p0 = spmm128(sup0, src, dst, edge_weight, z128)
    x1, sup1, base1 = _make_cmm(3, 128, 128, True)(p0[0], p0[1], base0, wc1, bf1)
    p1 = spmm128(sup1, src, dst, edge_weight, z128)
    sup2, base2 = _make_cmm(3, 128, 128, False)(p1[0], p1[1], base1, wc2, bf2)
    p2 = spmm128(sup2, src, dst, edge_weight, z128)
    ycat3, base3 = _make_cmm(4, 128, 64, False, True)(p2[0], p2[1], base2, x1,
                                                      wc3, bf3)
    p3 = _make_spmm(128, 64)(ycat3, src, dst, edge_weight, z128)
    return _make_out(64)(p3[0], p3[1], base3)
